# baseline (speedup 1.0000x reference)
import functools

import jax
import jax.numpy as jnp
from jax import lax
from jax.experimental import pallas as pl
from jax.experimental.pallas import tpu as pltpu
from jax.experimental.pallas import tpu_sc as plsc

_NC, _NS, _L = 2, 16, 16
_NW = _NC * _NS

_PRECEDENCE = ((42, 5), (47, 5), (94, 6), (43, 3), (45, 3), (60, 2), (62, 2), (61, 1))


@functools.lru_cache(maxsize=None)
def _make_encoder(n, n_rows, d):
    per_w = n // _NW
    n_sel = n_rows + 1
    sb = 64
    nsb = per_w // sb
    cchunk = 24
    gchunk = 16
    unroll = 4
    nj = d // (_L * unroll)
    ngrp = per_w // _L

    mesh = plsc.VectorSubcoreMesh(core_axis_name="c", subcore_axis_name="s")

    @functools.partial(
        pl.kernel,
        mesh=mesh,
        out_type=(
            jax.ShapeDtypeStruct((n, d), jnp.float32),
            jax.ShapeDtypeStruct((_NW * n_sel, d), jnp.float32),
        ),
        scratch_types=[
            pltpu.VMEM((n_sel, d), jnp.float32),
            pltpu.VMEM((per_w,), jnp.int32),
            pltpu.VMEM((per_w,), jnp.int32),
            pltpu.VMEM((ngrp, _L), jnp.int32),
            pltpu.VMEM((ngrp, _L), jnp.int32),
            pltpu.VMEM((2, cchunk, d), jnp.float32),
            pltpu.VMEM((2, gchunk, d), jnp.float32),
            pltpu.SemaphoreType.DMA,
            pltpu.SemaphoreType.DMA,
            pltpu.SemaphoreType.DMA,
            pltpu.SemaphoreType.DMA,
            pltpu.SemaphoreType.DMA,
            pltpu.SemaphoreType.DMA,
            pltpu.SemaphoreType.DMA,
        ],
    )
    def encode(tok_hbm, op_hbm, tab_hbm, out_hbm, stage_hbm,
               tab8_v, tok_v, op_v, idx_v, idxg_v, crows_v, grows_v,
               sem_in, c0, c1, g0, g1, w0, w1):
        wid = lax.axis_index("s") * _NC + lax.axis_index("c")
        base = wid * per_w
        wbase = wid * n_sel
        csems = (c0, c1)
        gsems = (g0, g1)
        wsems = (w0, w1)

        in_tok = pltpu.async_copy(tok_hbm.at[pl.ds(base, per_w)], tok_v, sem_in)
        in_op = pltpu.async_copy(op_hbm.at[pl.ds(base, per_w)], op_v, sem_in)
        pltpu.sync_copy(tab_hbm, tab8_v.at[pl.ds(0, n_rows)])
        zeros = jnp.zeros((_L,), jnp.float32)

        @plsc.parallel_loop(0, d // _L)
        def _scale(j):
            sl = pl.ds(j * _L, _L)
            for r in range(n_rows):
                tab8_v[r, sl] = tab8_v[r, sl] * jnp.float32(0.2)
            tab8_v[n_rows, sl] = zeros

        pltpu.sync_copy(tab8_v, stage_hbm.at[pl.ds(wbase, n_sel)])

        in_tok.wait()
        in_op.wait()

        @plsc.parallel_loop(0, ngrp)
        def _ibody(i):
            sl = pl.ds(i * _L, _L)
            t = tok_v[sl]
            o = op_v[sl]
            pid = jnp.zeros((_L,), jnp.int32)
            for tid, lvl in _PRECEDENCE:
                pid = jnp.where(t == tid, jnp.int32(lvl), pid)
            pid = jnp.where(o > 0, pid, jnp.int32(n_rows))
            idx_v[i, pl.ds(0, _L)] = pid
            idxg_v[i, pl.ds(0, _L)] = pid + wbase

        gstart0 = pltpu.async_copy(
            stage_hbm.at[idxg_v.at[3]], grows_v.at[0], gsems[0])
        del gstart0

        def construct(s, half, cb):
            g0v = idx_v[4 * s + half, pl.ds(0, _L)]
            g1v = idx_v[4 * s + half + 1, pl.ds(0, _L)]
            if half == 0:
                pids = [g0v[t] for t in range(_L)] + [g1v[t] for t in range(8)]
            else:
                pids = [g0v[t] for t in range(8, _L)] + [g1v[t] for t in range(_L)]

            @plsc.parallel_loop(0, nj, unroll=unroll)
            def jb(j):
                for u in range(unroll):
                    sl = pl.ds((j * unroll + u) * _L, _L)
                    for r0 in range(0, cchunk, 8):
                        vals = [tab8_v[pids[r0 + t], sl] for t in range(8)]
                        for t in range(8):
                            crows_v[cb, r0 + t, sl] = vals[t]

            pltpu.async_copy(
                crows_v.at[cb],
                out_hbm.at[pl.ds(base + s * sb + half * cchunk, cchunk)],
                csems[cb])

        def sb_body(s, gb):
            for cb in range(2):
                @pl.when(s >= 1)
                def _():
                    pltpu.make_async_copy(
                        out_hbm.at[pl.ds(base, cchunk)], crows_v.at[cb],
                        csems[cb]).wait()
                construct(s, cb, cb)
            @pl.when(s >= 1)
            def _():
                pltpu.make_async_copy(
                    out_hbm.at[pl.ds(base, gchunk)], grows_v.at[1 - gb],
                    wsems[1 - gb]).wait()
            @pl.when(s + 1 < nsb)
            def _():
                pltpu.async_copy(
                    stage_hbm.at[idxg_v.at[4 * (s + 1) + 3]],
                    grows_v.at[1 - gb], gsems[1 - gb])
            pltpu.make_async_copy(
                out_hbm.at[pl.ds(base, gchunk)], grows_v.at[gb],
                gsems[gb]).wait()
            pltpu.async_copy(
                grows_v.at[gb],
                out_hbm.at[pl.ds(base + s * sb + 2 * cchunk, gchunk)],
                wsems[gb])

        def pair_body(i, _):
            sb_body(2 * i, 0)
            sb_body(2 * i + 1, 1)
            return 0
        lax.fori_loop(0, nsb // 2, pair_body, 0)

        for cb in range(2):
            pltpu.make_async_copy(
                out_hbm.at[pl.ds(base, cchunk)], crows_v.at[cb],
                csems[cb]).wait()
        pltpu.make_async_copy(
            out_hbm.at[pl.ds(base, gchunk)], grows_v.at[(nsb - 1) & 1],
            wsems[(nsb - 1) & 1]).wait()

    return encode


def kernel(token_ids, operators, table):
    b, s = token_ids.shape
    n_rows, d = table.shape
    n = b * s
    tok = token_ids.reshape(n).astype(jnp.int32)
    ops = operators.reshape(n).astype(jnp.int32)
    out, _ = _make_encoder(n, n_rows, d)(tok, ops, table)
    return out.reshape(b, s, d)


# speedup vs baseline: 1.3624x; 1.3624x over previous
import functools

import jax
import jax.numpy as jnp
from jax import lax
from jax.experimental import pallas as pl
from jax.experimental.pallas import tpu as pltpu
from jax.experimental.pallas import tpu_sc as plsc

_NC, _NS, _L = 2, 16, 16
_NW = _NC * _NS

_PRECEDENCE = ((42, 5), (47, 5), (94, 6), (43, 3), (45, 3), (60, 2), (62, 2), (61, 1))


@functools.lru_cache(maxsize=None)
def _make_encoder(n, n_rows, d):
    per_w = n // _NW
    n_sel = n_rows + 1
    chunk = 32
    npairs = per_w // (2 * chunk)
    unroll = 4
    nj = d // (_L * unroll)

    mesh = plsc.VectorSubcoreMesh(core_axis_name="c", subcore_axis_name="s")

    @functools.partial(
        pl.kernel,
        mesh=mesh,
        out_type=jax.ShapeDtypeStruct((n, d), jnp.float32),
        scratch_types=[
            pltpu.VMEM((n_sel, d), jnp.float32),
            pltpu.VMEM((per_w,), jnp.int32),
            pltpu.VMEM((per_w,), jnp.int32),
            pltpu.VMEM((per_w // _L, _L), jnp.int32),
            pltpu.VMEM((2, chunk, d), jnp.float32),
            pltpu.SemaphoreType.DMA,
            pltpu.SemaphoreType.DMA,
            pltpu.SemaphoreType.DMA,
        ],
    )
    def encode(tok_hbm, op_hbm, tab_hbm, out_hbm,
               tab8_v, tok_v, op_v, idx_v, rows_v, sem_in, s0, s1):
        wid = lax.axis_index("s") * _NC + lax.axis_index("c")
        base = wid * per_w
        sems = (s0, s1)

        in_tok = pltpu.async_copy(tok_hbm.at[pl.ds(base, per_w)], tok_v, sem_in)
        in_op = pltpu.async_copy(op_hbm.at[pl.ds(base, per_w)], op_v, sem_in)
        pltpu.sync_copy(tab_hbm, tab8_v.at[pl.ds(0, n_rows)])
        zeros = jnp.zeros((_L,), jnp.float32)

        @plsc.parallel_loop(0, d // _L)
        def _scale(j):
            sl = pl.ds(j * _L, _L)
            for r in range(n_rows):
                tab8_v[r, sl] = tab8_v[r, sl] * jnp.float32(0.2)
            tab8_v[n_rows, sl] = zeros

        in_tok.wait()
        in_op.wait()
        @plsc.parallel_loop(0, per_w // _L)
        def _ibody(i):
            sl = pl.ds(i * _L, _L)
            t = tok_v[sl]
            o = op_v[sl]
            pid = jnp.zeros((_L,), jnp.int32)
            for tid, lvl in _PRECEDENCE:
                pid = jnp.where(t == tid, jnp.int32(lvl), pid)
            pid = jnp.where(o > 0, pid, jnp.int32(n_rows))
            idx_v[i, pl.ds(0, _L)] = pid

        def pair_body(i, _):
            for half in range(2):
                c = 2 * i + half
                b = half
                @pl.when(i >= 1)
                def _():
                    pltpu.make_async_copy(
                        out_hbm.at[pl.ds(base, chunk)], rows_v.at[b],
                        sems[b]).wait()
                pids = []
                for g in range(2):
                    vec = idx_v[2 * c + g, pl.ds(0, _L)]
                    for kk in range(_L):
                        pids.append(vec[kk])
                @plsc.parallel_loop(0, nj, unroll=4)
                def jb(j):
                    for u in range(unroll):
                        sl = pl.ds((j * unroll + u) * _L, _L)
                        for r0 in range(0, chunk, 8):
                            vals = [tab8_v[pids[r0 + t], sl] for t in range(8)]
                            for t in range(8):
                                rows_v[b, r0 + t, sl] = vals[t]
                pltpu.async_copy(
                    rows_v.at[b], out_hbm.at[pl.ds(base + c * chunk, chunk)],
                    sems[b])
            return 0
        lax.fori_loop(0, npairs, pair_body, 0)
        for b in range(2):
            pltpu.make_async_copy(
                out_hbm.at[pl.ds(base, chunk)], rows_v.at[b], sems[b]).wait()

    return encode


def kernel(token_ids, operators, table):
    b, s = token_ids.shape
    n_rows, d = table.shape
    n = b * s
    tok = token_ids.reshape(n).astype(jnp.int32)
    ops = operators.reshape(n).astype(jnp.int32)
    out = _make_encoder(n, n_rows, d)(tok, ops, table)
    return out.reshape(b, s, d)


# speedup vs baseline: 1.3644x; 1.0015x over previous
import functools

import jax
import jax.numpy as jnp
from jax import lax
from jax.experimental import pallas as pl
from jax.experimental.pallas import tpu as pltpu
from jax.experimental.pallas import tpu_sc as plsc

_NC, _NS, _L = 2, 16, 16
_NW = _NC * _NS

_PRECEDENCE = ((42, 5), (47, 5), (94, 6), (43, 3), (45, 3), (60, 2), (62, 2), (61, 1))


@functools.lru_cache(maxsize=None)
def _make_encoder(n, n_rows, d):
    per_w = n // _NW
    n_sel = n_rows + 1
    chunk = 16
    nbuf = 4
    ngroups = per_w // (nbuf * chunk)
    unroll = 4
    nj = d // (_L * unroll)

    mesh = plsc.VectorSubcoreMesh(core_axis_name="c", subcore_axis_name="s")

    @functools.partial(
        pl.kernel,
        mesh=mesh,
        out_type=jax.ShapeDtypeStruct((n, d), jnp.float32),
        scratch_types=[
            pltpu.VMEM((n_sel, d), jnp.float32),
            pltpu.VMEM((per_w,), jnp.int32),
            pltpu.VMEM((per_w,), jnp.int32),
            pltpu.VMEM((per_w // _L, _L), jnp.int32),
            pltpu.VMEM((nbuf, chunk, d), jnp.float32),
            pltpu.SemaphoreType.DMA,
            pltpu.SemaphoreType.DMA,
            pltpu.SemaphoreType.DMA,
            pltpu.SemaphoreType.DMA,
            pltpu.SemaphoreType.DMA,
        ],
    )
    def encode(tok_hbm, op_hbm, tab_hbm, out_hbm,
               tab8_v, tok_v, op_v, idx_v, rows_v, sem_in, s0, s1, s2, s3):
        wid = lax.axis_index("s") * _NC + lax.axis_index("c")
        base = wid * per_w
        sems = (s0, s1, s2, s3)

        in_tok = pltpu.async_copy(tok_hbm.at[pl.ds(base, per_w)], tok_v, sem_in)
        in_op = pltpu.async_copy(op_hbm.at[pl.ds(base, per_w)], op_v, sem_in)
        pltpu.sync_copy(tab_hbm, tab8_v.at[pl.ds(0, n_rows)])
        zeros = jnp.zeros((_L,), jnp.float32)

        @plsc.parallel_loop(0, d // _L)
        def _scale(j):
            sl = pl.ds(j * _L, _L)
            for r in range(n_rows):
                tab8_v[r, sl] = tab8_v[r, sl] * jnp.float32(0.2)
            tab8_v[n_rows, sl] = zeros

        in_tok.wait()
        in_op.wait()
        @plsc.parallel_loop(0, per_w // _L)
        def _ibody(i):
            sl = pl.ds(i * _L, _L)
            t = tok_v[sl]
            o = op_v[sl]
            pid = jnp.zeros((_L,), jnp.int32)
            for tid, lvl in _PRECEDENCE:
                pid = jnp.where(t == tid, jnp.int32(lvl), pid)
            pid = jnp.where(o > 0, pid, jnp.int32(n_rows))
            idx_v[i, pl.ds(0, _L)] = pid

        def grp_body(i, _):
            for b in range(nbuf):
                c = nbuf * i + b
                @pl.when(i >= 1)
                def _():
                    pltpu.make_async_copy(
                        out_hbm.at[pl.ds(base, chunk)], rows_v.at[b],
                        sems[b]).wait()
                vec = idx_v[c, pl.ds(0, _L)]
                pids = [vec[kk] for kk in range(_L)]
                @plsc.parallel_loop(0, nj, unroll=4)
                def jb(j):
                    for u in range(unroll):
                        sl = pl.ds((j * unroll + u) * _L, _L)
                        for r0 in range(0, chunk, 8):
                            vals = [tab8_v[pids[r0 + t], sl] for t in range(8)]
                            for t in range(8):
                                rows_v[b, r0 + t, sl] = vals[t]
                pltpu.async_copy(
                    rows_v.at[b], out_hbm.at[pl.ds(base + c * chunk, chunk)],
                    sems[b])
            return 0
        lax.fori_loop(0, ngroups, grp_body, 0)
        for b in range(nbuf):
            pltpu.make_async_copy(
                out_hbm.at[pl.ds(base, chunk)], rows_v.at[b], sems[b]).wait()

    return encode


def kernel(token_ids, operators, table):
    b, s = token_ids.shape
    n_rows, d = table.shape
    n = b * s
    tok = token_ids.reshape(n).astype(jnp.int32)
    ops = operators.reshape(n).astype(jnp.int32)
    out = _make_encoder(n, n_rows, d)(tok, ops, table)
    return out.reshape(b, s, d)
